# baseline (speedup 1.0000x reference)
import functools
import math

import jax
import jax.numpy as jnp
from jax import lax
from jax.experimental import pallas as pl
from jax.experimental.pallas import tpu as pltpu
from jax.experimental.pallas import tpu_sc as plsc

N = 10000
E = 320000
H = 128
NH = 8
DH = 16
NVF = 128
INNER = 512

NC = 2
NS = 16
NW = NC * NS
EPT = E // NW
CE = 80
NCH = EPT // CE
CF = 1000

@functools.cache
def _sc_mesh():
    return plsc.VectorSubcoreMesh(
        core_axis_name="c", subcore_axis_name="s",
        num_cores=NC, num_subcores=NS)


def _ln(x, eps=1e-6):
    mu = jnp.mean(x, axis=-1, keepdims=True)
    var = jnp.mean((x - mu) ** 2, axis=-1, keepdims=True)
    return (x - mu) * lax.rsqrt(var + eps)


def _silu(x):
    return x * jax.nn.sigmoid(x)


def _dot16(a, b):
    return jnp.dot(a.astype(jnp.bfloat16), b.astype(jnp.bfloat16),
                   preferred_element_type=jnp.float32)


def _gelu_tanh(x):
    return 0.5 * x * (1.0 + jnp.tanh(0.7978845608028654 * (x + 0.044715 * x * x * x)))


def _n1_body(x_ref, th_ref, wada_ref, bada_ref, wqkv_ref, mod_ref, q_ref,
             k_ref, v_ref):
    mod = jnp.dot(_silu(th_ref[...]), wada_ref[...],
                  preferred_element_type=jnp.float32) + bada_ref[...]
    mod_ref[...] = mod
    shift = mod[:, 0:H]
    scale = mod[:, H:2 * H]
    xm = _ln(x_ref[...]) * (1.0 + scale) + shift
    qkv = jnp.dot(xm, wqkv_ref[...], preferred_element_type=jnp.float32)
    q_ref[...] = qkv[:, 0:H]
    k_ref[...] = qkv[:, H:2 * H]
    v_ref[...] = qkv[:, 2 * H:3 * H]


def _tc_n1(x, t_emb_h, W_ada, b_ada, W_qkv):
    BN = 2000
    grid = (N // BN,)
    return pl.pallas_call(
        _n1_body,
        grid=grid,
        in_specs=[
            pl.BlockSpec((BN, H), lambda i: (i, 0)),
            pl.BlockSpec((BN, H), lambda i: (i, 0)),
            pl.BlockSpec((H, 6 * H), lambda i: (0, 0)),
            pl.BlockSpec((1, 6 * H), lambda i: (0, 0)),
            pl.BlockSpec((H, 3 * H), lambda i: (0, 0)),
        ],
        out_specs=[
            pl.BlockSpec((BN, 6 * H), lambda i: (i, 0)),
            pl.BlockSpec((BN, H), lambda i: (i, 0)),
            pl.BlockSpec((BN, H), lambda i: (i, 0)),
            pl.BlockSpec((BN, H), lambda i: (i, 0)),
        ],
        out_shape=[
            jax.ShapeDtypeStruct((N, 6 * H), jnp.float32),
            jax.ShapeDtypeStruct((N, H), jnp.float32),
            jax.ShapeDtypeStruct((N, H), jnp.float32),
            jax.ShapeDtypeStruct((N, H), jnp.float32),
        ],
    )(x, t_emb_h, W_ada, b_ada.reshape(1, 6 * H), W_qkv)


def _gather_combine_body(op, a_hbm, b_hbm, src_hbm, tgt_hbm, out_hbm,
                         sidx, tidx, av, bv):
    wid = lax.axis_index("s") * NC + lax.axis_index("c")

    def chunk(i, _):
        base = wid * EPT + i * CE
        pltpu.sync_copy(src_hbm.at[pl.ds(base, CE)], sidx)
        pltpu.sync_copy(tgt_hbm.at[pl.ds(base, CE)], tidx)
        pltpu.sync_copy(a_hbm.at[sidx], av)
        pltpu.sync_copy(b_hbm.at[tidx], bv)

        def row(r, _):
            for h in range(NH):
                sl = pl.ds(h * DH, DH)
                if op == "mul":
                    av[r, sl] = av[r, sl] * bv[r, sl]
                else:
                    av[r, sl] = av[r, sl] + bv[r, sl]
            return 0

        lax.fori_loop(0, CE, row, 0, unroll=2)
        pltpu.sync_copy(av, out_hbm.at[pl.ds(base, CE)])
        return 0

    lax.fori_loop(0, NCH, chunk, 0)


def _sc_gather_combine(op, table_a, table_b, src, tgt):
    body = functools.partial(_gather_combine_body, op)
    return pl.kernel(
        body,
        out_type=jax.ShapeDtypeStruct((E, H), jnp.float32),
        mesh=_sc_mesh(),
        scratch_types=[
            pltpu.VMEM((CE,), jnp.int32),
            pltpu.VMEM((CE,), jnp.int32),
            pltpu.VMEM((CE, H), jnp.float32),
            pltpu.VMEM((CE, H), jnp.float32),
        ],
    )(table_a, table_b, src, tgt)


def _e1_body(ea_ref, dist_ref, te_ref, t_ref, wadae_ref, badae_ref,
             wemb_ref, bemb_ref, wle0_ref, wle1_ref, w_ref, e16_ref):
    BE = ea_ref.shape[0]
    mod2 = _dot16(_silu(te_ref[...]), wadae_ref[...]) + badae_ref[...]
    e_shift = mod2[:, 0:H]
    e_scale = mod2[:, H:2 * H]
    ea = (_dot16(ea_ref[...], wemb_ref[0:H, :])
          + _dot16(dist_ref[...], wemb_ref[H:H + NVF, :])
          + bemb_ref[...])
    eam = _ln(ea) * (1.0 + e_scale) + e_shift
    e_attn = _gelu_tanh(_dot16(eam, wle0_ref[...]))
    le1 = _dot16(eam, wle1_ref[...])
    prod = t_ref[...] * e_attn
    gi = lax.broadcasted_iota(jnp.int32, (H, NH), 0) // DH
    gh = lax.broadcasted_iota(jnp.int32, (H, NH), 1)
    G = (gi == gh).astype(jnp.float32)
    alpha = jnp.dot(prod, G, preferred_element_type=jnp.float32) * (1.0 / math.sqrt(DH))
    e = jnp.exp(alpha)
    e16_ref[...] = jnp.concatenate([e, e], axis=1)
    w_ref[...] = le1 * jnp.dot(e, G.T, preferred_element_type=jnp.float32)
    del BE


def _tc_e1(edge_attr, dist, t_emb_e, T, W_ada_e, b_ada_e, W_edge_emb,
           b_edge_emb, W_le0, W_le1):
    BE = 2000
    grid = (E // BE,)
    return pl.pallas_call(
        _e1_body,
        grid=grid,
        in_specs=[
            pl.BlockSpec((BE, H), lambda i: (i, 0)),
            pl.BlockSpec((BE, NVF), lambda i: (i, 0)),
            pl.BlockSpec((BE, H), lambda i: (i, 0)),
            pl.BlockSpec((BE, H), lambda i: (i, 0)),
            pl.BlockSpec((H, 2 * H), lambda i: (0, 0)),
            pl.BlockSpec((1, 2 * H), lambda i: (0, 0)),
            pl.BlockSpec((H + NVF, H), lambda i: (0, 0)),
            pl.BlockSpec((1, H), lambda i: (0, 0)),
            pl.BlockSpec((H, H), lambda i: (0, 0)),
            pl.BlockSpec((H, H), lambda i: (0, 0)),
        ],
        out_specs=[
            pl.BlockSpec((BE, H), lambda i: (i, 0)),
            pl.BlockSpec((BE, DH), lambda i: (i, 0)),
        ],
        out_shape=[
            jax.ShapeDtypeStruct((E, H), jnp.float32),
            jax.ShapeDtypeStruct((E, DH), jnp.float32),
        ],
    )(edge_attr, dist, t_emb_e, T, W_ada_e[:, 0:2 * H],
      b_ada_e[0:2 * H].reshape(1, 2 * H), W_edge_emb,
      b_edge_emb.reshape(1, H), W_le0, W_le1)


ACC = N * NH + 2 * DH

def _scc_body(wf_hbm, tgt_hbm, S_out, acc, wv, tv):
    cid = lax.axis_index("c")
    sid = lax.axis_index("s")

    zeros16 = jnp.zeros((DH,), jnp.float32)

    def zstep(j, _):
        acc[pl.ds(j * DH, DH)] = zeros16
        return 0

    lax.fori_loop(0, ACC // DH, zstep, 0)
    lower8 = lax.iota(jnp.int32, DH) < NH

    def chunk(i, _):
        base = cid * (E // NC) + i * CF
        pltpu.sync_copy(tgt_hbm.at[pl.ds(base, CF)], tv.at[pl.ds(0, CF)])
        pltpu.sync_copy(wf_hbm.at[pl.ds(sid * (E * NH) + base * NH, CF * NH)],
                        wv.at[pl.ds(0, CF * NH)])

        def edge(j, _):
            tl = tv[pl.ds(j, DH)][0]
            pay = wv[pl.ds(j * NH, DH)]
            vsel = jnp.where(lower8, pay, 0.0)
            sl = pl.ds(DH + tl * NH, DH)
            acc[sl] = acc[sl] + vsel
            return 0

        lax.fori_loop(0, CF, edge, 0, unroll=2)
        return 0

    lax.fori_loop(0, E // NC // CF, chunk, 0)
    pltpu.sync_copy(acc.at[pl.ds(DH, N * NH)],
                    S_out.at[pl.ds((cid * NS + sid) * (N * NH), N * NH)])


def _sc_scatter_S(w_flat, tgt):
    return pl.kernel(
        _scc_body,
        out_type=jax.ShapeDtypeStruct((NC * NS * N * NH,), jnp.float32),
        mesh=_sc_mesh(),
        scratch_types=[
            pltpu.VMEM((ACC,), jnp.float32),
            pltpu.VMEM((CF * NH + DH,), jnp.float32),
            pltpu.VMEM((CF + DH,), jnp.int32),
        ],
    )(w_flat, tgt)


def _scs_body(ef_hbm, tgt_hbm, s_out, acc, ev, tv):
    wid = lax.axis_index("s") * NC + lax.axis_index("c")

    zeros16 = jnp.zeros((DH,), jnp.float32)

    def zstep(j, _):
        acc[pl.ds(j * DH, DH)] = zeros16
        return 0

    lax.fori_loop(0, ACC // DH, zstep, 0)
    lower8 = lax.iota(jnp.int32, DH) < NH

    def chunk(i, _):
        base = wid * EPT + i * CF
        pltpu.sync_copy(tgt_hbm.at[pl.ds(base, CF)], tv.at[pl.ds(0, CF)])
        pltpu.sync_copy(ef_hbm.at[pl.ds(base * DH, CF * DH)],
                        ev.at[pl.ds(0, CF * DH)])

        def edge(j, _):
            tl = tv[pl.ds(j, DH)][0]
            pay = ev[pl.ds(j * DH, DH)]
            vsel = jnp.where(lower8, pay, 0.0)
            sl = pl.ds(DH + tl * NH, DH)
            acc[sl] = acc[sl] + vsel
            return 0

        lax.fori_loop(0, CF, edge, 0, unroll=2)
        return 0

    lax.fori_loop(0, EPT // CF, chunk, 0)
    pltpu.sync_copy(acc.at[pl.ds(DH, N * NH)],
                    s_out.at[pl.ds(wid * (N * NH), N * NH)])


def _sc_scatter_s(e_flat, tgt):
    return pl.kernel(
        _scs_body,
        out_type=jax.ShapeDtypeStruct((NW * N * NH,), jnp.float32),
        mesh=_sc_mesh(),
        scratch_types=[
            pltpu.VMEM((ACC,), jnp.float32),
            pltpu.VMEM((CF * DH + DH,), jnp.float32),
            pltpu.VMEM((CF + DH,), jnp.int32),
        ],
    )(e_flat, tgt)


def _sred_body(sg_ref, out_ref):
    out_ref[...] = jnp.sum(sg_ref[...], axis=0, keepdims=True)


def _tc_sred(s_g):
    CHK = 16000
    grid = ((N * NH) // CHK,)
    return pl.pallas_call(
        _sred_body,
        grid=grid,
        in_specs=[pl.BlockSpec((NW, CHK), lambda i: (0, i))],
        out_specs=pl.BlockSpec((1, CHK), lambda i: (0, i)),
        out_shape=jax.ShapeDtypeStruct((1, N * NH), jnp.float32),
    )(s_g)


def _n2_body(x_ref, v_ref, S_ref, ss_ref, mod_ref, g2_ref, b2_ref,
             wn2e_ref, w1_ref, w3_ref, w2_ref, hout_ref, p_ref):
    S = S_ref[0] + S_ref[1]
    s8 = ss_ref[...]
    gi = lax.broadcasted_iota(jnp.int32, (NH, H), 1) // DH
    gh = lax.broadcasted_iota(jnp.int32, (NH, H), 0)
    GT = (gi == gh).astype(jnp.float32)
    denom = jnp.dot(s8, GT, preferred_element_type=jnp.float32) + 1e-16
    out = v_ref[...] * S / denom
    mod = mod_ref[...]
    gate_msa = mod[:, 2 * H:3 * H]
    shift_mlp = mod[:, 3 * H:4 * H]
    scale_mlp = mod[:, 4 * H:5 * H]
    gate_mlp = mod[:, 5 * H:6 * H]
    h_node = x_ref[...] + gate_msa * out
    h_node = (_ln(h_node) * g2_ref[...] + b2_ref[...]) * (1.0 + scale_mlp) + shift_mlp
    a = _silu(jnp.dot(h_node, w1_ref[...], preferred_element_type=jnp.float32))
    b = jnp.dot(h_node, w3_ref[...], preferred_element_type=jnp.float32)
    sw = jnp.dot(a * b, w2_ref[...], preferred_element_type=jnp.float32)
    hout_ref[...] = h_node + gate_mlp * sw
    p_ref[...] = jnp.dot(out, wn2e_ref[...], preferred_element_type=jnp.float32)


def _tc_n2(x, V, S2, ss2, mod, g2, b2, W_n2e, W1, W3, W2):
    BN = 2000
    grid = (N // BN,)
    return pl.pallas_call(
        _n2_body,
        grid=grid,
        in_specs=[
            pl.BlockSpec((BN, H), lambda i: (i, 0)),
            pl.BlockSpec((BN, H), lambda i: (i, 0)),
            pl.BlockSpec((NC, BN, H), lambda i: (0, i, 0)),
            pl.BlockSpec((BN, NH), lambda i: (i, 0)),
            pl.BlockSpec((BN, 6 * H), lambda i: (i, 0)),
            pl.BlockSpec((1, H), lambda i: (0, 0)),
            pl.BlockSpec((1, H), lambda i: (0, 0)),
            pl.BlockSpec((H, H), lambda i: (0, 0)),
            pl.BlockSpec((H, INNER), lambda i: (0, 0)),
            pl.BlockSpec((H, INNER), lambda i: (0, 0)),
            pl.BlockSpec((INNER, H), lambda i: (0, 0)),
        ],
        out_specs=[
            pl.BlockSpec((BN, H), lambda i: (i, 0)),
            pl.BlockSpec((BN, H), lambda i: (i, 0)),
        ],
        out_shape=[
            jax.ShapeDtypeStruct((N, H), jnp.float32),
            jax.ShapeDtypeStruct((N, H), jnp.float32),
        ],
    )(x, V, S2, ss2, mod, g2.reshape(1, H), b2.reshape(1, H), W_n2e, W1, W3, W2)


def _e2_body(eattr_ref, dist_ref, te_ref, hep_ref, wadae_ref, badae_ref,
             wemb_ref, bemb_ref, bn2e_ref, we1_ref, we3_ref, we2_ref,
             heo_ref):
    mod4 = _dot16(_silu(te_ref[...]), wadae_ref[...]) + badae_ref[...]
    e_gate_msa = mod4[:, 0:H]
    e_shift_mlp = mod4[:, H:2 * H]
    e_scale_mlp = mod4[:, 2 * H:3 * H]
    e_gate_mlp = mod4[:, 3 * H:4 * H]
    ea = (_dot16(eattr_ref[...], wemb_ref[0:H, :])
          + _dot16(dist_ref[...], wemb_ref[H:H + NVF, :])
          + bemb_ref[...])
    h_edge = hep_ref[...] + bn2e_ref[...]
    h_edge = eattr_ref[...] + e_gate_msa * h_edge
    h_edge = _ln(h_edge) * (1.0 + e_scale_mlp) + e_shift_mlp
    a = _silu(_dot16(h_edge, we1_ref[...]))
    b = _dot16(h_edge, we3_ref[...])
    sw = _dot16(a * b, we2_ref[...])
    heo_ref[...] = ea + h_edge + e_gate_mlp * sw


def _tc_e2(edge_attr, dist, t_emb_e, hep, W_ada_e, b_ada_e, W_edge_emb,
           b_edge_emb, b_n2e, We1, We3, We2):
    BE = 2000
    grid = (E // BE,)
    return pl.pallas_call(
        _e2_body,
        grid=grid,
        in_specs=[
            pl.BlockSpec((BE, H), lambda i: (i, 0)),
            pl.BlockSpec((BE, NVF), lambda i: (i, 0)),
            pl.BlockSpec((BE, H), lambda i: (i, 0)),
            pl.BlockSpec((BE, H), lambda i: (i, 0)),
            pl.BlockSpec((H, 4 * H), lambda i: (0, 0)),
            pl.BlockSpec((1, 4 * H), lambda i: (0, 0)),
            pl.BlockSpec((H + NVF, H), lambda i: (0, 0)),
            pl.BlockSpec((1, H), lambda i: (0, 0)),
            pl.BlockSpec((1, H), lambda i: (0, 0)),
            pl.BlockSpec((H, INNER), lambda i: (0, 0)),
            pl.BlockSpec((H, INNER), lambda i: (0, 0)),
            pl.BlockSpec((INNER, H), lambda i: (0, 0)),
        ],
        out_specs=[pl.BlockSpec((BE, H), lambda i: (i, 0))],
        out_shape=[jax.ShapeDtypeStruct((E, H), jnp.float32)],
    )(edge_attr, dist, t_emb_e, hep, W_ada_e[:, 2 * H:6 * H],
      b_ada_e[2 * H:6 * H].reshape(1, 4 * H), W_edge_emb,
      b_edge_emb.reshape(1, H), b_n2e.reshape(1, H), We1, We3, We2)[0]


def kernel(batch, x, t_emb_h, edge_attr, edge_index, t_emb_e, dist,
           W_edge_emb, b_edge_emb, W_ada, b_ada, W_ada_e, b_ada_e,
           W_qkv, W_le0, W_le1, W_n2e, b_n2e, g2, b2,
           W1, W3, W2, We1, We3, We2):
    del batch
    src = edge_index[0]
    tgt = edge_index[1]

    mod, Q, K, V = _tc_n1(x, t_emb_h, W_ada, b_ada, W_qkv)
    T = _sc_gather_combine("mul", Q, K, src, tgt)
    w, e16 = _tc_e1(edge_attr, dist, t_emb_e, T, W_ada_e, b_ada_e,
                    W_edge_emb, b_edge_emb, W_le0, W_le1)
    w_flat = jnp.transpose(w.reshape(E, NS, NH), (1, 0, 2)).reshape(NS * E * NH)
    e_flat = e16.reshape(E * DH)
    S_g = _sc_scatter_S(w_flat, tgt)
    s_g = _sc_scatter_s(e_flat, tgt)
    S2 = jnp.transpose(S_g.reshape(NC, NS, N, NH), (0, 2, 1, 3)).reshape(NC, N, H)
    ss2 = _tc_sred(s_g.reshape(NW, N * NH)).reshape(N, NH)
    h_out, P = _tc_n2(x, V, S2, ss2, mod, g2, b2, W_n2e, W1, W3, W2)
    hep = _sc_gather_combine("add", P, P, src, tgt)
    h_edge_out = _tc_e2(edge_attr, dist, t_emb_e, hep, W_ada_e, b_ada_e,
                        W_edge_emb, b_edge_emb, b_n2e, We1, We3, We2)
    return (h_out, h_edge_out)


# speedup vs baseline: 1.0129x; 1.0129x over previous
import functools
import math

import jax
import jax.numpy as jnp
from jax import lax
from jax.experimental import pallas as pl
from jax.experimental.pallas import tpu as pltpu
from jax.experimental.pallas import tpu_sc as plsc

N = 10000
E = 320000
H = 128
NH = 8
DH = 16
NVF = 128
INNER = 512

NC = 2
NS = 16
NW = NC * NS
EPT = E // NW
CE = 80
NCH = EPT // CE
CF = 1000

@functools.cache
def _sc_mesh():
    return plsc.VectorSubcoreMesh(
        core_axis_name="c", subcore_axis_name="s",
        num_cores=NC, num_subcores=NS)


def _ln(x, eps=1e-6):
    mu = jnp.mean(x, axis=-1, keepdims=True)
    var = jnp.mean((x - mu) ** 2, axis=-1, keepdims=True)
    return (x - mu) * lax.rsqrt(var + eps)


def _silu(x):
    return x * jax.nn.sigmoid(x)


def _gelu_tanh(x):
    return 0.5 * x * (1.0 + jnp.tanh(0.7978845608028654 * (x + 0.044715 * x * x * x)))


def _n1_body(x_ref, th_ref, wada_ref, bada_ref, wqkv_ref, mod_ref, q_ref,
             k_ref, v_ref):
    mod = jnp.dot(_silu(th_ref[...]), wada_ref[...],
                  preferred_element_type=jnp.float32) + bada_ref[...]
    mod_ref[...] = mod
    shift = mod[:, 0:H]
    scale = mod[:, H:2 * H]
    xm = _ln(x_ref[...]) * (1.0 + scale) + shift
    qkv = jnp.dot(xm, wqkv_ref[...], preferred_element_type=jnp.float32)
    q_ref[...] = qkv[:, 0:H]
    k_ref[...] = qkv[:, H:2 * H]
    v_ref[...] = qkv[:, 2 * H:3 * H]


def _tc_n1(x, t_emb_h, W_ada, b_ada, W_qkv):
    BN = 2000
    grid = (N // BN,)
    return pl.pallas_call(
        _n1_body,
        grid=grid,
        in_specs=[
            pl.BlockSpec((BN, H), lambda i: (i, 0)),
            pl.BlockSpec((BN, H), lambda i: (i, 0)),
            pl.BlockSpec((H, 6 * H), lambda i: (0, 0)),
            pl.BlockSpec((1, 6 * H), lambda i: (0, 0)),
            pl.BlockSpec((H, 3 * H), lambda i: (0, 0)),
        ],
        out_specs=[
            pl.BlockSpec((BN, 6 * H), lambda i: (i, 0)),
            pl.BlockSpec((BN, H), lambda i: (i, 0)),
            pl.BlockSpec((BN, H), lambda i: (i, 0)),
            pl.BlockSpec((BN, H), lambda i: (i, 0)),
        ],
        out_shape=[
            jax.ShapeDtypeStruct((N, 6 * H), jnp.float32),
            jax.ShapeDtypeStruct((N, H), jnp.float32),
            jax.ShapeDtypeStruct((N, H), jnp.float32),
            jax.ShapeDtypeStruct((N, H), jnp.float32),
        ],
    )(x, t_emb_h, W_ada, b_ada.reshape(1, 6 * H), W_qkv)


def _gather_combine_body(op, a_hbm, b_hbm, src_hbm, tgt_hbm, out_hbm,
                         sidx, tidx, av, bv):
    wid = lax.axis_index("s") * NC + lax.axis_index("c")

    def chunk(i, _):
        base = wid * EPT + i * CE
        pltpu.sync_copy(src_hbm.at[pl.ds(base, CE)], sidx)
        pltpu.sync_copy(tgt_hbm.at[pl.ds(base, CE)], tidx)
        pltpu.sync_copy(a_hbm.at[sidx], av)
        pltpu.sync_copy(b_hbm.at[tidx], bv)

        def row(r, _):
            for h in range(NH):
                sl = pl.ds(h * DH, DH)
                if op == "mul":
                    av[r, sl] = av[r, sl] * bv[r, sl]
                else:
                    av[r, sl] = av[r, sl] + bv[r, sl]
            return 0

        lax.fori_loop(0, CE, row, 0, unroll=2)
        pltpu.sync_copy(av, out_hbm.at[pl.ds(base, CE)])
        return 0

    lax.fori_loop(0, NCH, chunk, 0)


def _sc_gather_combine(op, table_a, table_b, src, tgt):
    body = functools.partial(_gather_combine_body, op)
    return pl.kernel(
        body,
        out_type=jax.ShapeDtypeStruct((E, H), jnp.float32),
        mesh=_sc_mesh(),
        scratch_types=[
            pltpu.VMEM((CE,), jnp.int32),
            pltpu.VMEM((CE,), jnp.int32),
            pltpu.VMEM((CE, H), jnp.float32),
            pltpu.VMEM((CE, H), jnp.float32),
        ],
    )(table_a, table_b, src, tgt)


def _e1_body(ea_ref, dist_ref, te_ref, t_ref, wadae_ref, badae_ref,
             wemb_ref, bemb_ref, wle0_ref, wle1_ref, w_ref, e16_ref):
    BE = ea_ref.shape[0]
    mod2 = jnp.dot(_silu(te_ref[...]), wadae_ref[...],
                   preferred_element_type=jnp.float32) + badae_ref[...]
    e_shift = mod2[:, 0:H]
    e_scale = mod2[:, H:2 * H]
    ea = (jnp.dot(ea_ref[...], wemb_ref[0:H, :],
                  preferred_element_type=jnp.float32)
          + jnp.dot(dist_ref[...], wemb_ref[H:H + NVF, :],
                    preferred_element_type=jnp.float32)
          + bemb_ref[...])
    eam = _ln(ea) * (1.0 + e_scale) + e_shift
    e_attn = _gelu_tanh(jnp.dot(eam, wle0_ref[...],
                                preferred_element_type=jnp.float32))
    le1 = jnp.dot(eam, wle1_ref[...], preferred_element_type=jnp.float32)
    prod = t_ref[...] * e_attn
    gi = lax.broadcasted_iota(jnp.int32, (H, NH), 0) // DH
    gh = lax.broadcasted_iota(jnp.int32, (H, NH), 1)
    G = (gi == gh).astype(jnp.float32)
    alpha = jnp.dot(prod, G, preferred_element_type=jnp.float32) * (1.0 / math.sqrt(DH))
    e = jnp.exp(alpha)
    e16_ref[...] = jnp.concatenate([e, e], axis=1)
    w_ref[...] = le1 * jnp.dot(e, G.T, preferred_element_type=jnp.float32)
    del BE


def _tc_e1(edge_attr, dist, t_emb_e, T, W_ada_e, b_ada_e, W_edge_emb,
           b_edge_emb, W_le0, W_le1):
    BE = 2000
    grid = (E // BE,)
    return pl.pallas_call(
        _e1_body,
        grid=grid,
        in_specs=[
            pl.BlockSpec((BE, H), lambda i: (i, 0)),
            pl.BlockSpec((BE, NVF), lambda i: (i, 0)),
            pl.BlockSpec((BE, H), lambda i: (i, 0)),
            pl.BlockSpec((BE, H), lambda i: (i, 0)),
            pl.BlockSpec((H, 2 * H), lambda i: (0, 0)),
            pl.BlockSpec((1, 2 * H), lambda i: (0, 0)),
            pl.BlockSpec((H + NVF, H), lambda i: (0, 0)),
            pl.BlockSpec((1, H), lambda i: (0, 0)),
            pl.BlockSpec((H, H), lambda i: (0, 0)),
            pl.BlockSpec((H, H), lambda i: (0, 0)),
        ],
        out_specs=[
            pl.BlockSpec((BE, H), lambda i: (i, 0)),
            pl.BlockSpec((BE, DH), lambda i: (i, 0)),
        ],
        out_shape=[
            jax.ShapeDtypeStruct((E, H), jnp.float32),
            jax.ShapeDtypeStruct((E, DH), jnp.float32),
        ],
    )(edge_attr, dist, t_emb_e, T, W_ada_e[:, 0:2 * H],
      b_ada_e[0:2 * H].reshape(1, 2 * H), W_edge_emb,
      b_edge_emb.reshape(1, H), W_le0, W_le1)


ACC = N * NH + 2 * DH

def _scc_body(wf_hbm, tgt_hbm, S_out, acc, wv, tv):
    cid = lax.axis_index("c")
    sid = lax.axis_index("s")

    zeros16 = jnp.zeros((DH,), jnp.float32)

    def zstep(j, _):
        acc[pl.ds(j * DH, DH)] = zeros16
        return 0

    lax.fori_loop(0, ACC // DH, zstep, 0)
    lower8 = lax.iota(jnp.int32, DH) < NH

    def chunk(i, _):
        base = cid * (E // NC) + i * CF
        pltpu.sync_copy(tgt_hbm.at[pl.ds(base, CF)], tv.at[pl.ds(0, CF)])
        pltpu.sync_copy(wf_hbm.at[pl.ds(sid * (E * NH) + base * NH, CF * NH)],
                        wv.at[pl.ds(0, CF * NH)])

        def edge(j, _):
            tl = tv[pl.ds(j, DH)][0]
            pay = wv[pl.ds(j * NH, DH)]
            vsel = jnp.where(lower8, pay, 0.0)
            sl = pl.ds(DH + tl * NH, DH)
            acc[sl] = acc[sl] + vsel
            return 0

        lax.fori_loop(0, CF, edge, 0, unroll=2)
        return 0

    lax.fori_loop(0, E // NC // CF, chunk, 0)
    pltpu.sync_copy(acc.at[pl.ds(DH, N * NH)],
                    S_out.at[pl.ds((cid * NS + sid) * (N * NH), N * NH)])


def _sc_scatter_S(w_flat, tgt):
    return pl.kernel(
        _scc_body,
        out_type=jax.ShapeDtypeStruct((NC * NS * N * NH,), jnp.float32),
        mesh=_sc_mesh(),
        scratch_types=[
            pltpu.VMEM((ACC,), jnp.float32),
            pltpu.VMEM((CF * NH + DH,), jnp.float32),
            pltpu.VMEM((CF + DH,), jnp.int32),
        ],
    )(w_flat, tgt)


def _scs_body(ef_hbm, tgt_hbm, s_out, acc, ev, tv):
    wid = lax.axis_index("s") * NC + lax.axis_index("c")

    zeros16 = jnp.zeros((DH,), jnp.float32)

    def zstep(j, _):
        acc[pl.ds(j * DH, DH)] = zeros16
        return 0

    lax.fori_loop(0, ACC // DH, zstep, 0)
    lower8 = lax.iota(jnp.int32, DH) < NH

    def chunk(i, _):
        base = wid * EPT + i * CF
        pltpu.sync_copy(tgt_hbm.at[pl.ds(base, CF)], tv.at[pl.ds(0, CF)])
        pltpu.sync_copy(ef_hbm.at[pl.ds(base * DH, CF * DH)],
                        ev.at[pl.ds(0, CF * DH)])

        def edge(j, _):
            tl = tv[pl.ds(j, DH)][0]
            pay = ev[pl.ds(j * DH, DH)]
            vsel = jnp.where(lower8, pay, 0.0)
            sl = pl.ds(DH + tl * NH, DH)
            acc[sl] = acc[sl] + vsel
            return 0

        lax.fori_loop(0, CF, edge, 0, unroll=2)
        return 0

    lax.fori_loop(0, EPT // CF, chunk, 0)
    pltpu.sync_copy(acc.at[pl.ds(DH, N * NH)],
                    s_out.at[pl.ds(wid * (N * NH), N * NH)])


def _sc_scatter_s(e_flat, tgt):
    return pl.kernel(
        _scs_body,
        out_type=jax.ShapeDtypeStruct((NW * N * NH,), jnp.float32),
        mesh=_sc_mesh(),
        scratch_types=[
            pltpu.VMEM((ACC,), jnp.float32),
            pltpu.VMEM((CF * DH + DH,), jnp.float32),
            pltpu.VMEM((CF + DH,), jnp.int32),
        ],
    )(e_flat, tgt)


def _sred_body(sg_ref, out_ref):
    out_ref[...] = jnp.sum(sg_ref[...], axis=0, keepdims=True)


def _tc_sred(s_g):
    CHK = 16000
    grid = ((N * NH) // CHK,)
    return pl.pallas_call(
        _sred_body,
        grid=grid,
        in_specs=[pl.BlockSpec((NW, CHK), lambda i: (0, i))],
        out_specs=pl.BlockSpec((1, CHK), lambda i: (0, i)),
        out_shape=jax.ShapeDtypeStruct((1, N * NH), jnp.float32),
    )(s_g)


def _n2_body(x_ref, v_ref, S_ref, ss_ref, mod_ref, g2_ref, b2_ref,
             wn2e_ref, w1_ref, w3_ref, w2_ref, hout_ref, p_ref):
    S = S_ref[0] + S_ref[1]
    s8 = ss_ref[...]
    gi = lax.broadcasted_iota(jnp.int32, (NH, H), 1) // DH
    gh = lax.broadcasted_iota(jnp.int32, (NH, H), 0)
    GT = (gi == gh).astype(jnp.float32)
    denom = jnp.dot(s8, GT, preferred_element_type=jnp.float32) + 1e-16
    out = v_ref[...] * S / denom
    mod = mod_ref[...]
    gate_msa = mod[:, 2 * H:3 * H]
    shift_mlp = mod[:, 3 * H:4 * H]
    scale_mlp = mod[:, 4 * H:5 * H]
    gate_mlp = mod[:, 5 * H:6 * H]
    h_node = x_ref[...] + gate_msa * out
    h_node = (_ln(h_node) * g2_ref[...] + b2_ref[...]) * (1.0 + scale_mlp) + shift_mlp
    a = _silu(jnp.dot(h_node, w1_ref[...], preferred_element_type=jnp.float32))
    b = jnp.dot(h_node, w3_ref[...], preferred_element_type=jnp.float32)
    sw = jnp.dot(a * b, w2_ref[...], preferred_element_type=jnp.float32)
    hout_ref[...] = h_node + gate_mlp * sw
    p_ref[...] = jnp.dot(out, wn2e_ref[...], preferred_element_type=jnp.float32)


def _tc_n2(x, V, S2, ss2, mod, g2, b2, W_n2e, W1, W3, W2):
    BN = 2000
    grid = (N // BN,)
    return pl.pallas_call(
        _n2_body,
        grid=grid,
        in_specs=[
            pl.BlockSpec((BN, H), lambda i: (i, 0)),
            pl.BlockSpec((BN, H), lambda i: (i, 0)),
            pl.BlockSpec((NC, BN, H), lambda i: (0, i, 0)),
            pl.BlockSpec((BN, NH), lambda i: (i, 0)),
            pl.BlockSpec((BN, 6 * H), lambda i: (i, 0)),
            pl.BlockSpec((1, H), lambda i: (0, 0)),
            pl.BlockSpec((1, H), lambda i: (0, 0)),
            pl.BlockSpec((H, H), lambda i: (0, 0)),
            pl.BlockSpec((H, INNER), lambda i: (0, 0)),
            pl.BlockSpec((H, INNER), lambda i: (0, 0)),
            pl.BlockSpec((INNER, H), lambda i: (0, 0)),
        ],
        out_specs=[
            pl.BlockSpec((BN, H), lambda i: (i, 0)),
            pl.BlockSpec((BN, H), lambda i: (i, 0)),
        ],
        out_shape=[
            jax.ShapeDtypeStruct((N, H), jnp.float32),
            jax.ShapeDtypeStruct((N, H), jnp.float32),
        ],
    )(x, V, S2, ss2, mod, g2.reshape(1, H), b2.reshape(1, H), W_n2e, W1, W3, W2)


def _e2_body(eattr_ref, dist_ref, te_ref, hep_ref, wadae_ref, badae_ref,
             wemb_ref, bemb_ref, bn2e_ref, we1_ref, we3_ref, we2_ref,
             heo_ref):
    mod4 = jnp.dot(_silu(te_ref[...]), wadae_ref[...],
                   preferred_element_type=jnp.float32) + badae_ref[...]
    e_gate_msa = mod4[:, 0:H]
    e_shift_mlp = mod4[:, H:2 * H]
    e_scale_mlp = mod4[:, 2 * H:3 * H]
    e_gate_mlp = mod4[:, 3 * H:4 * H]
    ea = (jnp.dot(eattr_ref[...], wemb_ref[0:H, :],
                  preferred_element_type=jnp.float32)
          + jnp.dot(dist_ref[...], wemb_ref[H:H + NVF, :],
                    preferred_element_type=jnp.float32)
          + bemb_ref[...])
    h_edge = hep_ref[...] + bn2e_ref[...]
    h_edge = eattr_ref[...] + e_gate_msa * h_edge
    h_edge = _ln(h_edge) * (1.0 + e_scale_mlp) + e_shift_mlp
    a = _silu(jnp.dot(h_edge, we1_ref[...], preferred_element_type=jnp.float32))
    b = jnp.dot(h_edge, we3_ref[...], preferred_element_type=jnp.float32)
    sw = jnp.dot(a * b, we2_ref[...], preferred_element_type=jnp.float32)
    heo_ref[...] = ea + h_edge + e_gate_mlp * sw


def _tc_e2(edge_attr, dist, t_emb_e, hep, W_ada_e, b_ada_e, W_edge_emb,
           b_edge_emb, b_n2e, We1, We3, We2):
    BE = 2000
    grid = (E // BE,)
    return pl.pallas_call(
        _e2_body,
        grid=grid,
        in_specs=[
            pl.BlockSpec((BE, H), lambda i: (i, 0)),
            pl.BlockSpec((BE, NVF), lambda i: (i, 0)),
            pl.BlockSpec((BE, H), lambda i: (i, 0)),
            pl.BlockSpec((BE, H), lambda i: (i, 0)),
            pl.BlockSpec((H, 4 * H), lambda i: (0, 0)),
            pl.BlockSpec((1, 4 * H), lambda i: (0, 0)),
            pl.BlockSpec((H + NVF, H), lambda i: (0, 0)),
            pl.BlockSpec((1, H), lambda i: (0, 0)),
            pl.BlockSpec((1, H), lambda i: (0, 0)),
            pl.BlockSpec((H, INNER), lambda i: (0, 0)),
            pl.BlockSpec((H, INNER), lambda i: (0, 0)),
            pl.BlockSpec((INNER, H), lambda i: (0, 0)),
        ],
        out_specs=[pl.BlockSpec((BE, H), lambda i: (i, 0))],
        out_shape=[jax.ShapeDtypeStruct((E, H), jnp.float32)],
    )(edge_attr, dist, t_emb_e, hep, W_ada_e[:, 2 * H:6 * H],
      b_ada_e[2 * H:6 * H].reshape(1, 4 * H), W_edge_emb,
      b_edge_emb.reshape(1, H), b_n2e.reshape(1, H), We1, We3, We2)[0]


def kernel(batch, x, t_emb_h, edge_attr, edge_index, t_emb_e, dist,
           W_edge_emb, b_edge_emb, W_ada, b_ada, W_ada_e, b_ada_e,
           W_qkv, W_le0, W_le1, W_n2e, b_n2e, g2, b2,
           W1, W3, W2, We1, We3, We2):
    del batch
    src = edge_index[0]
    tgt = edge_index[1]

    mod, Q, K, V = _tc_n1(x, t_emb_h, W_ada, b_ada, W_qkv)
    T = _sc_gather_combine("mul", Q, K, src, tgt)
    w, e16 = _tc_e1(edge_attr, dist, t_emb_e, T, W_ada_e, b_ada_e,
                    W_edge_emb, b_edge_emb, W_le0, W_le1)
    w_flat = jnp.transpose(w.reshape(E, NS, NH), (1, 0, 2)).reshape(NS * E * NH)
    e_flat = e16.reshape(E * DH)
    S_g = _sc_scatter_S(w_flat, tgt)
    s_g = _sc_scatter_s(e_flat, tgt)
    S2 = jnp.transpose(S_g.reshape(NC, NS, N, NH), (0, 2, 1, 3)).reshape(NC, N, H)
    ss2 = _tc_sred(s_g.reshape(NW, N * NH)).reshape(N, NH)
    h_out, P = _tc_n2(x, V, S2, ss2, mod, g2, b2, W_n2e, W1, W3, W2)
    hep = _sc_gather_combine("add", P, P, src, tgt)
    h_edge_out = _tc_e2(edge_attr, dist, t_emb_e, hep, W_ada_e, b_ada_e,
                        W_edge_emb, b_edge_emb, b_n2e, We1, We3, We2)
    return (h_out, h_edge_out)
